# Initial kernel scaffold; baseline (speedup 1.0000x reference)
import jax
import jax.numpy as jnp
from jax.experimental import pallas as pl


def kernel(x, edge_index, edge_weight, prev_hidden_state, Wz, bz, Wr, br, Wh, bh, Lz_W, Lz_b, Lr_W, Lr_b, Lh_W, Lh_b, lin_W, lin_b):
    raise NotImplementedError("write your pallas kernel here")


# speedup vs baseline: 40.4134x; 40.4134x over previous
import functools

import jax
import jax.numpy as jnp
from jax import lax
from jax.experimental import pallas as pl
from jax.experimental.pallas import tpu as pltpu
from jax.experimental.pallas import tpu_sc as plsc

NC = 2
NS = 16
NW = NC * NS
LANES = 16

BT = 2048
CH = 1024
CD = 7 * 1024


def _ceil_to(a, m):
    return (a + m - 1) // m * m


def _rsqrt16(d):
    i = lax.bitcast_convert_type(d, jnp.int32)
    i = 0x5F3759DF - lax.shift_right_logical(i, 1)
    y = lax.bitcast_convert_type(i, jnp.float32)
    for _ in range(3):
        y = y * (1.5 - 0.5 * d * y * y)
    return y


def _make_deg_kernel(epad, npad):
    epw = epad // NW
    ncd = epw // CD
    mesh = plsc.VectorSubcoreMesh(
        core_axis_name="c", subcore_axis_name="s",
        num_cores=NC, num_subcores=NS)
    params = pltpu.CompilerParams(
        use_tc_tiling_on_sc=False, needs_layout_passes=False)
    slice_n = npad // NS

    @functools.partial(
        pl.kernel,
        out_type=(jax.ShapeDtypeStruct((npad,), jnp.float32),
                  jax.ShapeDtypeStruct((npad,), jnp.float32)),
        mesh=mesh,
        scratch_types=[
            pltpu.VMEM((CD // 128, 128), jnp.int32),
            pltpu.VMEM((CD,), jnp.float32),
            pltpu.VMEM((slice_n,), jnp.float32),
            pltpu.VMEM_SHARED((npad,), jnp.float32),
        ],
        compiler_params=params,
    )
    def deg_kernel(col2_hbm, ew_hbm, out0_hbm, out1_hbm, col_v, ew_v, zb, acc):
        c = lax.axis_index("c")
        s = lax.axis_index("s")
        w = c * NS + s

        def zb_body(i, _):
            zb[pl.ds(pl.multiple_of(i * LANES, LANES), LANES)] = (
                jnp.zeros((LANES,), jnp.float32))
            return _
        lax.fori_loop(0, slice_n // LANES, zb_body, 0)
        r0 = pl.multiple_of(s * slice_n, 8)
        pltpu.sync_copy(zb, acc.at[pl.ds(r0, slice_n)])
        plsc.subcore_barrier()

        def chunk_body(k, _):
            base = pl.multiple_of(w * epw + k * CD, 8)
            rb = pl.multiple_of((w * epw + k * CD) // 128, 8)
            pltpu.sync_copy(col2_hbm.at[pl.ds(rb, CD // 128)], col_v)
            pltpu.sync_copy(ew_hbm.at[pl.ds(base, CD)], ew_v)

            def sub_body(j, _):
                o = pl.multiple_of(j * 128, 8)
                pltpu.sync_copy(ew_v.at[pl.ds(o, 128)],
                                acc.at[col_v.at[j]], add=True)
                return _
            lax.fori_loop(0, CD // 128, sub_body, 0)
            return _
        lax.fori_loop(0, ncd, chunk_body, 0)
        plsc.subcore_barrier()

        @pl.when(c == 0)
        def _():
            pltpu.sync_copy(acc.at[pl.ds(r0, slice_n)],
                            out0_hbm.at[pl.ds(r0, slice_n)])

        @pl.when(c == 1)
        def _():
            pltpu.sync_copy(acc.at[pl.ds(r0, slice_n)],
                            out1_hbm.at[pl.ds(r0, slice_n)])

    return deg_kernel


def _make_msg_kernel(epad, npad, lags):
    epw = epad // NW
    nch = epw // CH
    dc = 1600
    mesh = plsc.VectorSubcoreMesh(
        core_axis_name="c", subcore_axis_name="s",
        num_cores=NC, num_subcores=NS)
    params = pltpu.CompilerParams(
        use_tc_tiling_on_sc=False, needs_layout_passes=False)
    slice_n = npad // NS

    @functools.partial(
        pl.kernel,
        out_type=(jax.ShapeDtypeStruct((npad, lags), jnp.float32),
                  jax.ShapeDtypeStruct((npad, lags), jnp.float32)),
        mesh=mesh,
        scratch_types=[
            pltpu.VMEM((CH // 128, 128), jnp.int32),
            pltpu.VMEM((CH // 128, 128), jnp.int32),
            pltpu.VMEM((CH,), jnp.float32),
            pltpu.VMEM((CH,), jnp.float32),
            pltpu.VMEM((CH,), jnp.float32),
            pltpu.VMEM((CH,), jnp.float32),
            pltpu.VMEM((CH, lags), jnp.float32),
            pltpu.VMEM((dc,), jnp.float32),
            pltpu.VMEM((dc,), jnp.float32),
            pltpu.VMEM((dc,), jnp.float32),
            pltpu.VMEM_SHARED((npad,), jnp.float32),
            pltpu.VMEM_SHARED((npad, lags), jnp.float32),
            pltpu.SemaphoreType.DMA,
        ],
        compiler_params=params,
    )
    def msg_kernel(row2_hbm, col2_hbm, ew_hbm, x_hbm, degp0_hbm, degp1_hbm,
                   out0_hbm, out1_hbm,
                   row_v, col_v, ew_v, norm_v, dr_v, dcl_v, xr_v,
                   da_v, db_v, dv_v, dinv_sh, acc, sem):
        c = lax.axis_index("c")
        s = lax.axis_index("s")
        w = c * NS + s

        iota = lax.iota(jnp.int32, LANES)
        half = lax.shift_right_logical(iota, 3)
        lane8 = lax.bitwise_and(iota, 7)

        def zx_body(j, _):
            plsc.store_scatter(xr_v, [j * 2 + half, lane8],
                               jnp.zeros((LANES,), jnp.float32))
            return _
        lax.fori_loop(0, CH * lags // LANES, zx_body, 0)

        r0 = pl.multiple_of(s * slice_n, 8)
        nfull = slice_n // CH
        rem = slice_n - nfull * CH

        def zacc_body(i, _):
            o = pl.multiple_of(i * CH, 8)
            pltpu.sync_copy(xr_v, acc.at[pl.ds(r0 + o, CH)])
            return _
        lax.fori_loop(0, nfull, zacc_body, 0)
        if rem:
            pltpu.sync_copy(xr_v.at[pl.ds(0, rem)],
                            acc.at[pl.ds(r0 + nfull * CH, rem)])

        def dinv_chunk(kk, _):
            off = pl.multiple_of(r0 + kk * dc, 8)
            pltpu.sync_copy(degp0_hbm.at[pl.ds(off, dc)], da_v)
            pltpu.sync_copy(degp1_hbm.at[pl.ds(off, dc)], db_v)

            def dinv_vec(j, _):
                o = pl.multiple_of(j * LANES, LANES)
                d = 1.0 + da_v[pl.ds(o, LANES)] + db_v[pl.ds(o, LANES)]
                dv_v[pl.ds(o, LANES)] = _rsqrt16(d)
                return _
            lax.fori_loop(0, dc // LANES, dinv_vec, 0)
            pltpu.sync_copy(dv_v, dinv_sh.at[pl.ds(off, dc)])
            return _
        lax.fori_loop(0, slice_n // dc, dinv_chunk, 0)

        plsc.subcore_barrier()

        nsub = CH // 128

        def edge_chunk(k, _):
            base = pl.multiple_of(w * epw + k * CH, 8)
            rb = pl.multiple_of((w * epw + k * CH) // 128, 8)
            pltpu.sync_copy(row2_hbm.at[pl.ds(rb, nsub)], row_v)
            pltpu.sync_copy(col2_hbm.at[pl.ds(rb, nsub)], col_v)
            pltpu.sync_copy(ew_hbm.at[pl.ds(base, CH)], ew_v)

            for j in range(nsub):
                o = j * 128
                pltpu.async_copy(
                    dinv_sh.at[row_v.at[j]], dr_v.at[pl.ds(o, 128)],
                    sem).wait()
                pltpu.async_copy(
                    dinv_sh.at[col_v.at[j]], dcl_v.at[pl.ds(o, 128)],
                    sem).wait()
                pltpu.async_copy(
                    x_hbm.at[row_v.at[j]], xr_v.at[pl.ds(o, 128)],
                    sem).wait()

            def norm_body(j, _):
                o = pl.multiple_of(j * LANES, LANES)
                norm_v[pl.ds(o, LANES)] = (
                    dr_v[pl.ds(o, LANES)] * ew_v[pl.ds(o, LANES)]
                    * dcl_v[pl.ds(o, LANES)])
                return _
            lax.fori_loop(0, CH // LANES, norm_body, 0)

            def scale_body(j, _):
                ir = j * 2 + half
                nv = plsc.load_gather(norm_v, [ir])
                xv = plsc.load_gather(xr_v, [ir, lane8])
                plsc.store_scatter(xr_v, [ir, lane8], xv * nv)
                return _
            lax.fori_loop(0, CH * lags // LANES, scale_body, 0)

            for j in range(nsub):
                pltpu.sync_copy(xr_v.at[pl.ds(j * 128, 128)],
                                acc.at[col_v.at[j]], add=True)
            return _
        lax.fori_loop(0, nch, edge_chunk, 0)

        plsc.subcore_barrier()

        @pl.when(c == 0)
        def _():
            pltpu.sync_copy(acc.at[pl.ds(r0, slice_n)],
                            out0_hbm.at[pl.ds(r0, slice_n)])

        @pl.when(c == 1)
        def _():
            pltpu.sync_copy(acc.at[pl.ds(r0, slice_n)],
                            out1_hbm.at[pl.ds(r0, slice_n)])

    return msg_kernel


def _tc_body(d0_ref, d1_ref, sxp0_ref, sxp1_ref, x_ref, h_ref,
             az_ref, bz_ref, cz_ref, ar_ref, br_ref, cr_ref,
             ah_ref, bh_ref, chh_ref, lwt_ref, lb_ref,
             y_ref, hn_ref):
    deg = 1.0 + d0_ref[...] + d1_ref[...]
    sx = sxp0_ref[...] + sxp1_ref[...] + x_ref[...] / deg
    h = h_ref[...]
    dot = lambda a, b: jnp.dot(a, b, preferred_element_type=jnp.float32)
    z = jax.nn.sigmoid(dot(sx, az_ref[...]) + dot(h, bz_ref[...])
                       + cz_ref[...])
    r = jax.nn.sigmoid(dot(sx, ar_ref[...]) + dot(h, br_ref[...])
                       + cr_ref[...])
    ht = jnp.tanh(dot(sx, ah_ref[...]) + dot(h * r, bh_ref[...])
                  + chh_ref[...])
    hn = z * h + (1.0 - z) * ht
    hn_ref[...] = hn
    y_ref[...] = jnp.sum(jnp.maximum(hn, 0.0) * lwt_ref[...],
                         axis=1, keepdims=True) + lb_ref[...]


def kernel(x, edge_index, edge_weight, prev_hidden_state,
           Wz, bz, Wr, br, Wh, bh,
           Lz_W, Lz_b, Lr_W, Lr_b, Lh_W, Lh_b,
           lin_W, lin_b):
    n, lags = x.shape
    f = Wz.shape[1]
    e = edge_weight.shape[0]

    npad = _ceil_to(n, 51200)
    epad = _ceil_to(e, NW * CD)

    row = edge_index[0]
    col = edge_index[1]
    pe = epad - e
    row2 = jnp.pad(row, (0, pe)).reshape(epad // 128, 128)
    col2 = jnp.pad(col, (0, pe)).reshape(epad // 128, 128)
    ewp = jnp.pad(edge_weight, (0, pe))
    xp = jnp.pad(x, ((0, npad - n), (0, 0)))
    hp = jnp.pad(prev_hidden_state, ((0, npad - n), (0, 0)))

    degp0, degp1 = _make_deg_kernel(epad, npad)(col2, ewp)
    sxp0, sxp1 = _make_msg_kernel(epad, npad, lags)(
        row2, col2, ewp, xp, degp0, degp1)

    az = Wz @ Lz_W[:f]
    bz_l = Lz_W[f:]
    cz = (bz @ Lz_W[:f] + Lz_b).reshape(1, f)
    ar = Wr @ Lr_W[:f]
    br_l = Lr_W[f:]
    cr = (br @ Lr_W[:f] + Lr_b).reshape(1, f)
    ah = Wh @ Lh_W[:f]
    bh_l = Lh_W[f:]
    ch = (bh @ Lh_W[:f] + Lh_b).reshape(1, f)
    lwt = lin_W.reshape(1, f)
    lb = lin_b.reshape(1, 1)

    grid = (npad // BT,)
    full = lambda shape: pl.BlockSpec(shape, lambda i: (0,) * len(shape))
    y_pad, hn_pad = pl.pallas_call(
        _tc_body,
        grid=grid,
        in_specs=[
            pl.BlockSpec((BT, 1), lambda i: (i, 0)),
            pl.BlockSpec((BT, 1), lambda i: (i, 0)),
            pl.BlockSpec((BT, lags), lambda i: (i, 0)),
            pl.BlockSpec((BT, lags), lambda i: (i, 0)),
            pl.BlockSpec((BT, lags), lambda i: (i, 0)),
            pl.BlockSpec((BT, f), lambda i: (i, 0)),
            full((lags, f)), full((f, f)), full((1, f)),
            full((lags, f)), full((f, f)), full((1, f)),
            full((lags, f)), full((f, f)), full((1, f)),
            full((1, f)), full((1, 1)),
        ],
        out_specs=[
            pl.BlockSpec((BT, 1), lambda i: (i, 0)),
            pl.BlockSpec((BT, f), lambda i: (i, 0)),
        ],
        out_shape=[
            jax.ShapeDtypeStruct((npad, 1), jnp.float32),
            jax.ShapeDtypeStruct((npad, f), jnp.float32),
        ],
    )(degp0.reshape(npad, 1), degp1.reshape(npad, 1), sxp0, sxp1,
      xp, hp, az, bz_l, cz, ar, br_l, cr, ah, bh_l, ch, lwt, lb)

    return (y_pad[:n], hn_pad[:n])


# speedup vs baseline: 55.1929x; 1.3657x over previous
import functools

import jax
import jax.numpy as jnp
from jax import lax
from jax.experimental import pallas as pl
from jax.experimental.pallas import tpu as pltpu
from jax.experimental.pallas import tpu_sc as plsc

NC = 2
NS = 16
NW = NC * NS
LANES = 16

BT = 2048
CH = 1024
CD = 7 * 1024


def _ceil_to(a, m):
    return (a + m - 1) // m * m


def _rsqrt16(d):
    i = lax.bitcast_convert_type(d, jnp.int32)
    i = 0x5F3759DF - lax.shift_right_logical(i, 1)
    y = lax.bitcast_convert_type(i, jnp.float32)
    for _ in range(3):
        y = y * (1.5 - 0.5 * d * y * y)
    return y


def _make_deg_kernel(epad, npad):
    epw = epad // NW
    ncd = epw // CD
    mesh = plsc.VectorSubcoreMesh(
        core_axis_name="c", subcore_axis_name="s",
        num_cores=NC, num_subcores=NS)
    params = pltpu.CompilerParams(
        use_tc_tiling_on_sc=False, needs_layout_passes=False)
    slice_n = npad // NS

    @functools.partial(
        pl.kernel,
        out_type=(jax.ShapeDtypeStruct((npad,), jnp.float32),
                  jax.ShapeDtypeStruct((npad,), jnp.float32)),
        mesh=mesh,
        scratch_types=[
            pltpu.VMEM((CD // 128, 128), jnp.int32),
            pltpu.VMEM((CD,), jnp.float32),
            pltpu.VMEM((slice_n,), jnp.float32),
            pltpu.VMEM_SHARED((npad,), jnp.float32),
        ],
        compiler_params=params,
    )
    def deg_kernel(col2_hbm, ew_hbm, out0_hbm, out1_hbm, col_v, ew_v, zb, acc):
        c = lax.axis_index("c")
        s = lax.axis_index("s")
        w = c * NS + s

        def zb_body(i, _):
            zb[pl.ds(pl.multiple_of(i * LANES, LANES), LANES)] = (
                jnp.zeros((LANES,), jnp.float32))
            return _
        lax.fori_loop(0, slice_n // LANES, zb_body, 0)
        r0 = pl.multiple_of(s * slice_n, 8)
        pltpu.sync_copy(zb, acc.at[pl.ds(r0, slice_n)])
        plsc.subcore_barrier()

        def chunk_body(k, _):
            base = pl.multiple_of(w * epw + k * CD, 8)
            rb = pl.multiple_of((w * epw + k * CD) // 128, 8)
            pltpu.sync_copy(col2_hbm.at[pl.ds(rb, CD // 128)], col_v)
            pltpu.sync_copy(ew_hbm.at[pl.ds(base, CD)], ew_v)

            def sub_body(j, _):
                o = pl.multiple_of(j * 128, 8)
                pltpu.sync_copy(ew_v.at[pl.ds(o, 128)],
                                acc.at[col_v.at[j]], add=True)
                return _
            lax.fori_loop(0, CD // 128, sub_body, 0)
            return _
        lax.fori_loop(0, ncd, chunk_body, 0)
        plsc.subcore_barrier()

        @pl.when(c == 0)
        def _():
            pltpu.sync_copy(acc.at[pl.ds(r0, slice_n)],
                            out0_hbm.at[pl.ds(r0, slice_n)])

        @pl.when(c == 1)
        def _():
            pltpu.sync_copy(acc.at[pl.ds(r0, slice_n)],
                            out1_hbm.at[pl.ds(r0, slice_n)])

    return deg_kernel


def _make_msg_kernel(epad, npad, lags):
    epw = epad // NW
    nch = epw // CH
    dc = 800
    mesh = plsc.VectorSubcoreMesh(
        core_axis_name="c", subcore_axis_name="s",
        num_cores=NC, num_subcores=NS)
    params = pltpu.CompilerParams(
        use_tc_tiling_on_sc=False, needs_layout_passes=False)
    slice_n = npad // NS

    @functools.partial(
        pl.kernel,
        out_type=(jax.ShapeDtypeStruct((npad, lags), jnp.float32),
                  jax.ShapeDtypeStruct((npad, lags), jnp.float32)),
        mesh=mesh,
        scratch_types=[
            pltpu.VMEM((CH // 128, 128), jnp.int32),
            pltpu.VMEM((CH // 128, 128), jnp.int32),
            pltpu.VMEM((CH,), jnp.float32),
            pltpu.VMEM((CH, lags), jnp.float32),
            pltpu.VMEM((dc,), jnp.float32),
            pltpu.VMEM((dc,), jnp.float32),
            pltpu.VMEM((dc,), jnp.float32),
            pltpu.VMEM((dc, lags), jnp.float32),
            pltpu.VMEM_SHARED((npad, lags), jnp.float32),
            pltpu.VMEM_SHARED((npad, lags), jnp.float32),
            pltpu.SemaphoreType.DMA,
        ],
        compiler_params=params,
    )
    def msg_kernel(row2_hbm, col2_hbm, ew_hbm, x_hbm, degp0_hbm, degp1_hbm,
                   out0_hbm, out1_hbm,
                   row_v, col_v, ew_v, xr_v,
                   da_v, db_v, dv_v, xst_v, xs_sh, acc, sem):
        c = lax.axis_index("c")
        s = lax.axis_index("s")
        w = c * NS + s

        iota = lax.iota(jnp.int32, LANES)
        half = lax.shift_right_logical(iota, 3)
        lane8 = lax.bitwise_and(iota, 7)

        def zx_body(j, _):
            plsc.store_scatter(xr_v, [j * 2 + half, lane8],
                               jnp.zeros((LANES,), jnp.float32))
            return _
        lax.fori_loop(0, CH * lags // LANES, zx_body, 0)

        r0 = pl.multiple_of(s * slice_n, 8)
        nfull = slice_n // CH
        rem = slice_n - nfull * CH

        def zacc_body(i, _):
            o = pl.multiple_of(i * CH, 8)
            pltpu.sync_copy(xr_v, acc.at[pl.ds(r0 + o, CH)])
            return _
        lax.fori_loop(0, nfull, zacc_body, 0)
        if rem:
            pltpu.sync_copy(xr_v.at[pl.ds(0, rem)],
                            acc.at[pl.ds(r0 + nfull * CH, rem)])

        def xs_chunk(kk, _):
            off = pl.multiple_of(r0 + kk * dc, 8)
            pltpu.sync_copy(degp0_hbm.at[pl.ds(off, dc)], da_v)
            pltpu.sync_copy(degp1_hbm.at[pl.ds(off, dc)], db_v)
            pltpu.sync_copy(x_hbm.at[pl.ds(off, dc)], xst_v)

            def dinv_vec(j, _):
                o = pl.multiple_of(j * LANES, LANES)
                d = 1.0 + da_v[pl.ds(o, LANES)] + db_v[pl.ds(o, LANES)]
                dv_v[pl.ds(o, LANES)] = _rsqrt16(d)
                return _
            lax.fori_loop(0, dc // LANES, dinv_vec, 0)

            def xsc_vec(i, _):
                ir = i * 2 + half
                dvv = plsc.load_gather(dv_v, [ir])
                xv = plsc.load_gather(xst_v, [ir, lane8])
                plsc.store_scatter(xst_v, [ir, lane8], xv * dvv)
                return _
            lax.fori_loop(0, dc * lags // LANES, xsc_vec, 0)

            pltpu.sync_copy(xst_v, xs_sh.at[pl.ds(off, dc)])
            return _
        lax.fori_loop(0, slice_n // dc, xs_chunk, 0)

        plsc.subcore_barrier()

        nsub = CH // 128
        per16s = 128 * lags // LANES

        def edge_chunk(k, _):
            base = pl.multiple_of(w * epw + k * CH, 8)
            rb = pl.multiple_of((w * epw + k * CH) // 128, 8)
            pltpu.sync_copy(row2_hbm.at[pl.ds(rb, nsub)], row_v)
            pltpu.sync_copy(col2_hbm.at[pl.ds(rb, nsub)], col_v)
            pltpu.sync_copy(ew_hbm.at[pl.ds(base, CH)], ew_v)

            for j in range(nsub):
                o = j * 128
                pltpu.async_copy(xs_sh.at[row_v.at[j]],
                                 xr_v.at[pl.ds(o, 128)], sem).wait()

                def scale_body(i, _):
                    ir = o + i * 2 + half
                    nv = plsc.load_gather(ew_v, [ir])
                    xv = plsc.load_gather(xr_v, [ir, lane8])
                    plsc.store_scatter(xr_v, [ir, lane8], xv * nv)
                    return _
                lax.fori_loop(0, per16s, scale_body, 0)

                pltpu.sync_copy(xr_v.at[pl.ds(o, 128)],
                                acc.at[col_v.at[j]], add=True)
            return _
        lax.fori_loop(0, nch, edge_chunk, 0)

        plsc.subcore_barrier()

        @pl.when(c == 0)
        def _():
            pltpu.sync_copy(acc.at[pl.ds(r0, slice_n)],
                            out0_hbm.at[pl.ds(r0, slice_n)])

        @pl.when(c == 1)
        def _():
            pltpu.sync_copy(acc.at[pl.ds(r0, slice_n)],
                            out1_hbm.at[pl.ds(r0, slice_n)])

    return msg_kernel


def _tc_body(d0_ref, d1_ref, sxp0_ref, sxp1_ref, x_ref, h_ref,
             az_ref, bz_ref, cz_ref, ar_ref, br_ref, cr_ref,
             ah_ref, bh_ref, chh_ref, lwt_ref, lb_ref,
             y_ref, hn_ref):
    deg = 1.0 + d0_ref[...] + d1_ref[...]
    dinv = jax.lax.rsqrt(deg)
    sx = dinv * (sxp0_ref[...] + sxp1_ref[...]) + x_ref[...] / deg
    h = h_ref[...]
    dot = lambda a, b: jnp.dot(a, b, preferred_element_type=jnp.float32)
    z = jax.nn.sigmoid(dot(sx, az_ref[...]) + dot(h, bz_ref[...])
                       + cz_ref[...])
    r = jax.nn.sigmoid(dot(sx, ar_ref[...]) + dot(h, br_ref[...])
                       + cr_ref[...])
    ht = jnp.tanh(dot(sx, ah_ref[...]) + dot(h * r, bh_ref[...])
                  + chh_ref[...])
    hn = z * h + (1.0 - z) * ht
    hn_ref[...] = hn
    y_ref[...] = jnp.sum(jnp.maximum(hn, 0.0) * lwt_ref[...],
                         axis=1, keepdims=True) + lb_ref[...]


def kernel(x, edge_index, edge_weight, prev_hidden_state,
           Wz, bz, Wr, br, Wh, bh,
           Lz_W, Lz_b, Lr_W, Lr_b, Lh_W, Lh_b,
           lin_W, lin_b):
    n, lags = x.shape
    f = Wz.shape[1]
    e = edge_weight.shape[0]

    npad = _ceil_to(n, 51200)
    epad = _ceil_to(e, NW * CD)

    row = edge_index[0]
    col = edge_index[1]
    pe = epad - e
    row2 = jnp.pad(row, (0, pe)).reshape(epad // 128, 128)
    col2 = jnp.pad(col, (0, pe)).reshape(epad // 128, 128)
    ewp = jnp.pad(edge_weight, (0, pe))
    xp = jnp.pad(x, ((0, npad - n), (0, 0)))
    hp = jnp.pad(prev_hidden_state, ((0, npad - n), (0, 0)))

    degp0, degp1 = _make_deg_kernel(epad, npad)(col2, ewp)
    sxp0, sxp1 = _make_msg_kernel(epad, npad, lags)(
        row2, col2, ewp, xp, degp0, degp1)

    az = Wz @ Lz_W[:f]
    bz_l = Lz_W[f:]
    cz = (bz @ Lz_W[:f] + Lz_b).reshape(1, f)
    ar = Wr @ Lr_W[:f]
    br_l = Lr_W[f:]
    cr = (br @ Lr_W[:f] + Lr_b).reshape(1, f)
    ah = Wh @ Lh_W[:f]
    bh_l = Lh_W[f:]
    ch = (bh @ Lh_W[:f] + Lh_b).reshape(1, f)
    lwt = lin_W.reshape(1, f)
    lb = lin_b.reshape(1, 1)

    grid = (npad // BT,)
    full = lambda shape: pl.BlockSpec(shape, lambda i: (0,) * len(shape))
    y_pad, hn_pad = pl.pallas_call(
        _tc_body,
        grid=grid,
        in_specs=[
            pl.BlockSpec((BT, 1), lambda i: (i, 0)),
            pl.BlockSpec((BT, 1), lambda i: (i, 0)),
            pl.BlockSpec((BT, lags), lambda i: (i, 0)),
            pl.BlockSpec((BT, lags), lambda i: (i, 0)),
            pl.BlockSpec((BT, lags), lambda i: (i, 0)),
            pl.BlockSpec((BT, f), lambda i: (i, 0)),
            full((lags, f)), full((f, f)), full((1, f)),
            full((lags, f)), full((f, f)), full((1, f)),
            full((lags, f)), full((f, f)), full((1, f)),
            full((1, f)), full((1, 1)),
        ],
        out_specs=[
            pl.BlockSpec((BT, 1), lambda i: (i, 0)),
            pl.BlockSpec((BT, f), lambda i: (i, 0)),
        ],
        out_shape=[
            jax.ShapeDtypeStruct((npad, 1), jnp.float32),
            jax.ShapeDtypeStruct((npad, f), jnp.float32),
        ],
    )(degp0.reshape(npad, 1), degp1.reshape(npad, 1), sxp0, sxp1,
      xp, hp, az, bz_l, cz, ar, br_l, cr, ah, bh_l, ch, lwt, lb)

    return (y_pad[:n], hn_pad[:n])


# speedup vs baseline: 58.8704x; 1.0666x over previous
import functools

import jax
import jax.numpy as jnp
from jax import lax
from jax.experimental import pallas as pl
from jax.experimental.pallas import tpu as pltpu
from jax.experimental.pallas import tpu_sc as plsc

NC = 2
NS = 16
NW = NC * NS
LANES = 16

BT = 2048
CH = 1024
CD = 7 * 1024


def _ceil_to(a, m):
    return (a + m - 1) // m * m


def _rsqrt16(d):
    i = lax.bitcast_convert_type(d, jnp.int32)
    i = 0x5F3759DF - lax.shift_right_logical(i, 1)
    y = lax.bitcast_convert_type(i, jnp.float32)
    for _ in range(3):
        y = y * (1.5 - 0.5 * d * y * y)
    return y


def _make_deg_kernel(epad, npad):
    epw = epad // NW
    ncd = epw // CD
    mesh = plsc.VectorSubcoreMesh(
        core_axis_name="c", subcore_axis_name="s",
        num_cores=NC, num_subcores=NS)
    params = pltpu.CompilerParams(
        use_tc_tiling_on_sc=False, needs_layout_passes=False)
    slice_n = npad // NS

    @functools.partial(
        pl.kernel,
        out_type=(jax.ShapeDtypeStruct((npad,), jnp.float32),
                  jax.ShapeDtypeStruct((npad,), jnp.float32)),
        mesh=mesh,
        scratch_types=[
            pltpu.VMEM((CD // 128, 128), jnp.int32),
            pltpu.VMEM((CD,), jnp.float32),
            pltpu.VMEM((slice_n,), jnp.float32),
            pltpu.VMEM_SHARED((npad,), jnp.float32),
        ],
        compiler_params=params,
    )
    def deg_kernel(col2_hbm, ew_hbm, out0_hbm, out1_hbm, col_v, ew_v, zb, acc):
        c = lax.axis_index("c")
        s = lax.axis_index("s")
        w = c * NS + s

        def zb_body(i, _):
            zb[pl.ds(pl.multiple_of(i * LANES, LANES), LANES)] = (
                jnp.zeros((LANES,), jnp.float32))
            return _
        lax.fori_loop(0, slice_n // LANES, zb_body, 0)
        r0 = pl.multiple_of(s * slice_n, 8)
        pltpu.sync_copy(zb, acc.at[pl.ds(r0, slice_n)])
        plsc.subcore_barrier()

        def chunk_body(k, _):
            base = pl.multiple_of(w * epw + k * CD, 8)
            rb = pl.multiple_of((w * epw + k * CD) // 128, 8)
            pltpu.sync_copy(col2_hbm.at[pl.ds(rb, CD // 128)], col_v)
            pltpu.sync_copy(ew_hbm.at[pl.ds(base, CD)], ew_v)

            def sub_body(j, _):
                o = pl.multiple_of(j * 128, 8)
                pltpu.sync_copy(ew_v.at[pl.ds(o, 128)],
                                acc.at[col_v.at[j]], add=True)
                return _
            lax.fori_loop(0, CD // 128, sub_body, 0)
            return _
        lax.fori_loop(0, ncd, chunk_body, 0)
        plsc.subcore_barrier()

        @pl.when(c == 0)
        def _():
            pltpu.sync_copy(acc.at[pl.ds(r0, slice_n)],
                            out0_hbm.at[pl.ds(r0, slice_n)])

        @pl.when(c == 1)
        def _():
            pltpu.sync_copy(acc.at[pl.ds(r0, slice_n)],
                            out1_hbm.at[pl.ds(r0, slice_n)])

    return deg_kernel


def _make_msg_kernel(epad, npad, lags):
    epw = epad // NW
    nch = epw // CH
    dc = 800
    mesh = plsc.VectorSubcoreMesh(
        core_axis_name="c", subcore_axis_name="s",
        num_cores=NC, num_subcores=NS)
    params = pltpu.CompilerParams(
        use_tc_tiling_on_sc=False, needs_layout_passes=False)
    slice_n = npad // NS

    @functools.partial(
        pl.kernel,
        out_type=(jax.ShapeDtypeStruct((npad, lags), jnp.float32),
                  jax.ShapeDtypeStruct((npad, lags), jnp.float32)),
        mesh=mesh,
        scratch_types=[
            pltpu.VMEM((CH,), jnp.int32),
            pltpu.VMEM((CH,), jnp.int32),
            pltpu.VMEM((CH,), jnp.float32),
            pltpu.VMEM((CH, lags), jnp.float32),
            pltpu.VMEM((dc,), jnp.float32),
            pltpu.VMEM((dc,), jnp.float32),
            pltpu.VMEM((dc,), jnp.float32),
            pltpu.VMEM((dc, lags), jnp.float32),
            pltpu.VMEM_SHARED((npad, lags), jnp.float32),
            pltpu.VMEM_SHARED((npad, lags), jnp.float32),
            pltpu.SemaphoreType.DMA,
        ],
        compiler_params=params,
    )
    def msg_kernel(row_hbm, col_hbm, ew_hbm, x_hbm, degp0_hbm, degp1_hbm,
                   out0_hbm, out1_hbm,
                   row_v, col_v, ew_v, xr_v,
                   da_v, db_v, dv_v, xst_v, xs_sh, acc, sem):
        c = lax.axis_index("c")
        s = lax.axis_index("s")
        w = c * NS + s

        iota = lax.iota(jnp.int32, LANES)
        half = lax.shift_right_logical(iota, 3)
        lane8 = lax.bitwise_and(iota, 7)

        def zx_body(j, _):
            plsc.store_scatter(xr_v, [j * 2 + half, lane8],
                               jnp.zeros((LANES,), jnp.float32))
            return _
        lax.fori_loop(0, CH * lags // LANES, zx_body, 0)

        r0 = pl.multiple_of(s * slice_n, 8)
        nfull = slice_n // CH
        rem = slice_n - nfull * CH

        def zacc_body(i, _):
            o = pl.multiple_of(i * CH, 8)
            pltpu.sync_copy(xr_v, acc.at[pl.ds(r0 + o, CH)])
            return _
        lax.fori_loop(0, nfull, zacc_body, 0)
        if rem:
            pltpu.sync_copy(xr_v.at[pl.ds(0, rem)],
                            acc.at[pl.ds(r0 + nfull * CH, rem)])

        def xs_chunk(kk, _):
            off = pl.multiple_of(r0 + kk * dc, 8)
            pltpu.sync_copy(degp0_hbm.at[pl.ds(off, dc)], da_v)
            pltpu.sync_copy(degp1_hbm.at[pl.ds(off, dc)], db_v)
            pltpu.sync_copy(x_hbm.at[pl.ds(off, dc)], xst_v)

            def dinv_vec(j, _):
                o = pl.multiple_of(j * LANES, LANES)
                d = 1.0 + da_v[pl.ds(o, LANES)] + db_v[pl.ds(o, LANES)]
                dv_v[pl.ds(o, LANES)] = _rsqrt16(d)
                return _
            lax.fori_loop(0, dc // LANES, dinv_vec, 0)

            def xsc_vec(i, _):
                ir = i * 2 + half
                dvv = plsc.load_gather(dv_v, [ir])
                xv = plsc.load_gather(xst_v, [ir, lane8])
                plsc.store_scatter(xst_v, [ir, lane8], xv * dvv)
                return _
            lax.fori_loop(0, dc * lags // LANES, xsc_vec, 0)

            pltpu.sync_copy(xst_v, xs_sh.at[pl.ds(off, dc)])
            return _
        lax.fori_loop(0, slice_n // dc, xs_chunk, 0)

        plsc.subcore_barrier()

        def edge_chunk(k, _):
            base = pl.multiple_of(w * epw + k * CH, 8)
            pltpu.sync_copy(row_hbm.at[pl.ds(base, CH)], row_v)
            pltpu.sync_copy(col_hbm.at[pl.ds(base, CH)], col_v)
            pltpu.sync_copy(ew_hbm.at[pl.ds(base, CH)], ew_v)

            pltpu.async_copy(xs_sh.at[row_v], xr_v, sem).wait()

            def scale_body(i, _):
                ir = i * 2 + half
                nv = plsc.load_gather(ew_v, [ir])
                xv = plsc.load_gather(xr_v, [ir, lane8])
                plsc.store_scatter(xr_v, [ir, lane8], xv * nv)
                return _
            lax.fori_loop(0, CH * lags // LANES, scale_body, 0)

            pltpu.sync_copy(xr_v, acc.at[col_v], add=True)
            return _
        lax.fori_loop(0, nch, edge_chunk, 0)

        plsc.subcore_barrier()

        @pl.when(c == 0)
        def _():
            pltpu.sync_copy(acc.at[pl.ds(r0, slice_n)],
                            out0_hbm.at[pl.ds(r0, slice_n)])

        @pl.when(c == 1)
        def _():
            pltpu.sync_copy(acc.at[pl.ds(r0, slice_n)],
                            out1_hbm.at[pl.ds(r0, slice_n)])

    return msg_kernel


def _tc_body(d0_ref, d1_ref, sxp0_ref, sxp1_ref, x_ref, h_ref,
             az_ref, bz_ref, cz_ref, ar_ref, br_ref, cr_ref,
             ah_ref, bh_ref, chh_ref, lwt_ref, lb_ref,
             y_ref, hn_ref):
    deg = 1.0 + d0_ref[...] + d1_ref[...]
    dinv = jax.lax.rsqrt(deg)
    sx = dinv * (sxp0_ref[...] + sxp1_ref[...]) + x_ref[...] / deg
    h = h_ref[...]
    dot = lambda a, b: jnp.dot(a, b, preferred_element_type=jnp.float32)
    z = jax.nn.sigmoid(dot(sx, az_ref[...]) + dot(h, bz_ref[...])
                       + cz_ref[...])
    r = jax.nn.sigmoid(dot(sx, ar_ref[...]) + dot(h, br_ref[...])
                       + cr_ref[...])
    ht = jnp.tanh(dot(sx, ah_ref[...]) + dot(h * r, bh_ref[...])
                  + chh_ref[...])
    hn = z * h + (1.0 - z) * ht
    hn_ref[...] = hn
    y_ref[...] = jnp.sum(jnp.maximum(hn, 0.0) * lwt_ref[...],
                         axis=1, keepdims=True) + lb_ref[...]


def kernel(x, edge_index, edge_weight, prev_hidden_state,
           Wz, bz, Wr, br, Wh, bh,
           Lz_W, Lz_b, Lr_W, Lr_b, Lh_W, Lh_b,
           lin_W, lin_b):
    n, lags = x.shape
    f = Wz.shape[1]
    e = edge_weight.shape[0]

    npad = _ceil_to(n, 51200)
    epad = _ceil_to(e, NW * CD)

    row = edge_index[0]
    col = edge_index[1]
    pe = epad - e
    rowp = jnp.pad(row, (0, pe))
    colp = jnp.pad(col, (0, pe))
    col2 = colp.reshape(epad // 128, 128)
    ewp = jnp.pad(edge_weight, (0, pe))
    xp = jnp.pad(x, ((0, npad - n), (0, 0)))
    hp = jnp.pad(prev_hidden_state, ((0, npad - n), (0, 0)))

    degp0, degp1 = _make_deg_kernel(epad, npad)(col2, ewp)
    sxp0, sxp1 = _make_msg_kernel(epad, npad, lags)(
        rowp, colp, ewp, xp, degp0, degp1)

    az = Wz @ Lz_W[:f]
    bz_l = Lz_W[f:]
    cz = (bz @ Lz_W[:f] + Lz_b).reshape(1, f)
    ar = Wr @ Lr_W[:f]
    br_l = Lr_W[f:]
    cr = (br @ Lr_W[:f] + Lr_b).reshape(1, f)
    ah = Wh @ Lh_W[:f]
    bh_l = Lh_W[f:]
    ch = (bh @ Lh_W[:f] + Lh_b).reshape(1, f)
    lwt = lin_W.reshape(1, f)
    lb = lin_b.reshape(1, 1)

    grid = (npad // BT,)
    full = lambda shape: pl.BlockSpec(shape, lambda i: (0,) * len(shape))
    y_pad, hn_pad = pl.pallas_call(
        _tc_body,
        grid=grid,
        in_specs=[
            pl.BlockSpec((BT, 1), lambda i: (i, 0)),
            pl.BlockSpec((BT, 1), lambda i: (i, 0)),
            pl.BlockSpec((BT, lags), lambda i: (i, 0)),
            pl.BlockSpec((BT, lags), lambda i: (i, 0)),
            pl.BlockSpec((BT, lags), lambda i: (i, 0)),
            pl.BlockSpec((BT, f), lambda i: (i, 0)),
            full((lags, f)), full((f, f)), full((1, f)),
            full((lags, f)), full((f, f)), full((1, f)),
            full((lags, f)), full((f, f)), full((1, f)),
            full((1, f)), full((1, 1)),
        ],
        out_specs=[
            pl.BlockSpec((BT, 1), lambda i: (i, 0)),
            pl.BlockSpec((BT, f), lambda i: (i, 0)),
        ],
        out_shape=[
            jax.ShapeDtypeStruct((npad, 1), jnp.float32),
            jax.ShapeDtypeStruct((npad, f), jnp.float32),
        ],
    )(degp0.reshape(npad, 1), degp1.reshape(npad, 1), sxp0, sxp1,
      xp, hp, az, bz_l, cz, ar, br_l, cr, ah, bh_l, ch, lwt, lb)

    return (y_pad[:n], hn_pad[:n])
